# baseline (speedup 1.0000x reference)
import functools

import jax
import jax.numpy as jnp
from jax import lax
from jax.experimental import pallas as pl
from jax.experimental.pallas import tpu as pltpu
from jax.experimental.pallas import tpu_sc as plsc

MAXLEN = 200
EMB = 64
LANES = 128
NUM_TILES = 32
NGBUF = 4
SPLIT = (104, 96)


def _tok_pos_embed(x_flat, tok_padded, pos_table):
    total = x_flat.shape[0]
    rows_per_tile = total // NUM_TILES
    nchunk = 2 * (rows_per_tile // MAXLEN)
    mesh = plsc.VectorSubcoreMesh(core_axis_name="c", subcore_axis_name="s")

    @functools.partial(
        pl.kernel,
        out_type=jax.ShapeDtypeStruct((total, EMB), jnp.float32),
        mesh=mesh,
        scratch_types=[
            pltpu.VMEM((rows_per_tile,), jnp.int32),
            pltpu.VMEM((MAXLEN, EMB), jnp.float32),
        ] + [pltpu.VMEM((SPLIT[b % 2], LANES), jnp.float32) for b in range(NGBUF)]
          + [pltpu.VMEM((SPLIT[t], EMB), jnp.float32) for t in range(2)]
          + [pltpu.SemaphoreType.DMA for _ in range(NGBUF + 2)],
    )
    def k(x_hbm, tok_hbm, pos_hbm, out_hbm, idx_v, pos_v, *bufs_and_sems):
        gbufs = bufs_and_sems[:NGBUF]
        tbufs = bufs_and_sems[NGBUF:NGBUF + 2]
        gsems = bufs_and_sems[NGBUF + 2:2 * NGBUF + 2]
        osems = bufs_and_sems[2 * NGBUF + 2:]
        wid = lax.axis_index("s") * 2 + lax.axis_index("c")
        base = wid * rows_per_tile
        pltpu.sync_copy(x_hbm.at[pl.ds(base, rows_per_tile)], idx_v)
        pltpu.sync_copy(pos_hbm, pos_v)

        def chunk_off(c):
            return (c // 2) * MAXLEN + (c % 2) * SPLIT[0]

        def issue_gather(c, b):
            pltpu.async_copy(
                tok_hbm.at[idx_v.at[pl.ds(chunk_off(c), SPLIT[b % 2])]],
                gbufs[b], gsems[b])

        def wait_gather(c, b):
            pltpu.make_async_copy(
                tok_hbm.at[idx_v.at[pl.ds(chunk_off(c), SPLIT[b % 2])]],
                gbufs[b], gsems[b]).wait()

        def issue_out(c, t):
            pltpu.async_copy(
                tbufs[t], out_hbm.at[pl.ds(base + chunk_off(c), SPLIT[t])],
                osems[t])

        def wait_out(c, t):
            pltpu.make_async_copy(
                tbufs[t], out_hbm.at[pl.ds(base + chunk_off(c), SPLIT[t])],
                osems[t]).wait()

        issue_gather(0, 0)
        issue_gather(1, 1)

        @pl.loop(0, nchunk, step=NGBUF)
        def _grp(g):
            for b in range(NGBUF):
                c = g + b
                t = b % 2
                n = SPLIT[t]
                p0 = t * SPLIT[0]
                bp = (b + 2) % NGBUF
                wait_gather(c, b)

                @pl.when(c + 2 < nchunk)
                def _prefetch():
                    issue_gather(c + 2, bp)

                @pl.when(c >= 2)
                def _drain():
                    wait_out(c - 2, t)

                @pl.loop(0, n, unroll=4)
                def _row(r):
                    for col in range(0, EMB, 16):
                        tbufs[t][r, pl.ds(col, 16)] = (
                            gbufs[b][r, pl.ds(col, 16)]
                            + pos_v[p0 + r, pl.ds(col, 16)])

                issue_out(c, t)

        wait_out(nchunk - 2, 0)
        wait_out(nchunk - 1, 1)

    return k(x_flat, tok_padded, pos_table)


def kernel(x, token_table, pos_table):
    batch, seq = x.shape
    if seq < MAXLEN:
        x = jnp.pad(x, ((0, 0), (0, MAXLEN - seq)))
    else:
        x = x[:, :MAXLEN]
    x_flat = x.reshape(-1).astype(jnp.int32)
    tok_padded = jnp.pad(token_table, ((0, 0), (0, LANES - EMB)))
    out = _tok_pos_embed(x_flat, tok_padded, pos_table)
    return out.reshape(batch, MAXLEN, EMB)
